# Initial kernel scaffold; baseline (speedup 1.0000x reference)
import functools

import jax
import jax.numpy as jnp
from jax import lax
from jax.experimental import pallas as pl
from jax.experimental.pallas import tpu as pltpu
from jax.experimental.pallas import tpu_sc as plsc

N = 10000
E = 320000
NODE_DIM = 128
EDGE_DIM = 16
H = 64
LAYERS = 4

NC = 2
NS = 16
NW = NC * NS
C = 128
NCHUNK = E // C
CHUNKS_LO = NCHUNK // NW
CHUNKS_REM = NCHUNK - CHUNKS_LO * NW
ROWS_PER_TILE = N // NS

_LRELU = 0.01


def _embed_body(nf, wn, bn, wm0, h_out, hw_out):
    h = jnp.dot(nf[...], wn[...], preferred_element_type=jnp.float32) + bn[...]
    h_out[...] = h
    hw_out[...] = jnp.dot(h, wm0[...], preferred_element_type=jnp.float32)


def _node_embed(node_feats, W_node, b_node, Wm0):
    blk = 2500
    grid = N // blk
    return pl.pallas_call(
        _embed_body,
        grid=(grid,),
        in_specs=[
            pl.BlockSpec((blk, NODE_DIM), lambda i: (i, 0)),
            pl.BlockSpec((NODE_DIM, H), lambda i: (0, 0)),
            pl.BlockSpec((1, H), lambda i: (0, 0)),
            pl.BlockSpec((H, H), lambda i: (0, 0)),
        ],
        out_specs=[
            pl.BlockSpec((blk, H), lambda i: (i, 0)),
            pl.BlockSpec((blk, H), lambda i: (i, 0)),
        ],
        out_shape=[
            jax.ShapeDtypeStruct((N, H), jnp.float32),
            jax.ShapeDtypeStruct((N, H), jnp.float32),
        ],
    )(node_feats, W_node, b_node.reshape(1, H), Wm0)


def _edge_body(ef, weff, beff, o0, o1, o2, o3):
    y = jnp.dot(ef[...], weff[...], preferred_element_type=jnp.float32) + beff[...]
    o0[...] = y[:, 0 * H:1 * H]
    o1[...] = y[:, 1 * H:2 * H]
    o2[...] = y[:, 2 * H:3 * H]
    o3[...] = y[:, 3 * H:4 * H]


def _edge_terms(edge_feats, Weff, beff):
    blk = 8000
    grid = E // blk
    return pl.pallas_call(
        _edge_body,
        grid=(grid,),
        in_specs=[
            pl.BlockSpec((blk, EDGE_DIM), lambda i: (i, 0)),
            pl.BlockSpec((EDGE_DIM, LAYERS * H), lambda i: (0, 0)),
            pl.BlockSpec((1, LAYERS * H), lambda i: (0, 0)),
        ],
        out_specs=[pl.BlockSpec((blk, H), lambda i: (i, 0))] * LAYERS,
        out_shape=[jax.ShapeDtypeStruct((E, H), jnp.float32)] * LAYERS,
    )(edge_feats, Weff, beff.reshape(1, LAYERS * H))


def _update_body(h, agg, wu, bu, wm_next, h_out, hw_out):
    a = agg[0] + agg[1]
    x = jnp.dot(h[...] + a, wu[...], preferred_element_type=jnp.float32) + bu[...]
    hn = jnp.maximum(x, x * _LRELU)
    h_out[...] = hn
    hw_out[...] = jnp.dot(hn, wm_next[...], preferred_element_type=jnp.float32)


def _node_update(h, agg, Wu_l, bu_l, Wm_next):
    return pl.pallas_call(
        _update_body,
        out_shape=[
            jax.ShapeDtypeStruct((N, H), jnp.float32),
            jax.ShapeDtypeStruct((N, H), jnp.float32),
        ],
    )(h, agg, Wu_l, bu_l.reshape(1, H), Wm_next)


def _final_body(h, agg, wu, bu, watt, wl1, bl1, wl2, bl2, out):
    a = agg[0] + agg[1]
    x = jnp.dot(h[...] + a, wu[...], preferred_element_type=jnp.float32) + bu[...]
    hn = jnp.maximum(x, x * _LRELU)
    logits = jnp.dot(hn, watt[...], preferred_element_type=jnp.float32)
    m = jnp.max(logits)
    p = jnp.exp(logits - m)
    attn = p / jnp.sum(p)
    sup = jnp.sum(attn * hn, axis=0, keepdims=True)
    z = jnp.dot(sup, wl1[...], preferred_element_type=jnp.float32) + bl1[...]
    z = jnp.maximum(z, 0.0)
    out[...] = jnp.dot(z, wl2[...], preferred_element_type=jnp.float32) + bl2[...]


def _final_head(h, agg, Wu_l, bu_l, w_att, Wl1, bl1, Wl2, bl2):
    return pl.pallas_call(
        _final_body,
        out_shape=jax.ShapeDtypeStruct((1, 1), jnp.float32),
    )(h, agg, Wu_l, bu_l.reshape(1, H), w_att.reshape(H, 1),
      Wl1, bl1.reshape(1, H), Wl2, bl2.reshape(1, 1))


def _sc_msg_body(hw_hbm, src_hbm, dst_hbm, ew_hbm, zeros_hbm, agg_hbm,
                 src_v, dst_v, rows_v, ew_v, agg_sh, sem_g, sem_e):
    cid = lax.axis_index("c")
    sid = lax.axis_index("s")
    wid = sid * NC + cid

    pltpu.sync_copy(zeros_hbm.at[pl.ds(sid * ROWS_PER_TILE, ROWS_PER_TILE)],
                    agg_sh.at[pl.ds(sid * ROWS_PER_TILE, ROWS_PER_TILE)])
    plsc.subcore_barrier()

    nloc = jnp.where(wid < CHUNKS_REM, CHUNKS_LO + 1, CHUNKS_LO)

    def chunk_body(i, carry):
        base = (wid + i * NW) * C
        pltpu.sync_copy(src_hbm.at[pl.ds(base, C)], src_v)
        pltpu.sync_copy(dst_hbm.at[pl.ds(base, C)], dst_v)
        ge = pltpu.async_copy(ew_hbm.at[pl.ds(base, C)], ew_v, sem_e)
        gg = pltpu.async_copy(hw_hbm.at[src_v], rows_v, sem_g)
        ge.wait()
        gg.wait()

        def row_body(r, rc):
            for k in range(H // 16):
                sl = pl.ds(k * 16, 16)
                x = rows_v[r, sl] + ew_v[r, sl]
                rows_v[r, sl] = jnp.maximum(x, x * _LRELU)
            return rc
        lax.fori_loop(0, C, row_body, 0)

        pltpu.sync_copy(rows_v, agg_sh.at[dst_v], add=True)
        return carry

    lax.fori_loop(0, nloc, chunk_body, 0)

    plsc.subcore_barrier()
    pltpu.sync_copy(agg_sh.at[pl.ds(sid * ROWS_PER_TILE, ROWS_PER_TILE)],
                    agg_hbm.at[cid, pl.ds(sid * ROWS_PER_TILE, ROWS_PER_TILE)])


_sc_msg = pl.kernel(
    _sc_msg_body,
    out_type=jax.ShapeDtypeStruct((NC, N, H), jnp.float32),
    mesh=plsc.VectorSubcoreMesh(core_axis_name="c", subcore_axis_name="s",
                                num_cores=NC, num_subcores=NS),
    scratch_types=[
        pltpu.VMEM((C,), jnp.int32),
        pltpu.VMEM((C,), jnp.int32),
        pltpu.VMEM((C, H), jnp.float32),
        pltpu.VMEM((C, H), jnp.float32),
        pltpu.VMEM_SHARED((N, H), jnp.float32),
        pltpu.SemaphoreType.DMA,
        pltpu.SemaphoreType.DMA,
    ],
)


def kernel(graph, node_feats, edge_feats, W_node, b_node, W_edge, b_edge,
           Wm, bm, Wu, bu, w_att, Wl1, bl1, Wl2, bl2):
    src = graph[0]
    dst = graph[1]

    Weff = jnp.concatenate([W_edge @ Wm[l] for l in range(LAYERS)], axis=1)
    beff = jnp.concatenate([b_edge @ Wm[l] + bm[l] for l in range(LAYERS)])

    h, hw = _node_embed(node_feats, W_node, b_node, Wm[0])
    ew = _edge_terms(edge_feats, Weff, beff)
    zeros = jnp.zeros((N, H), jnp.float32)

    out = None
    for l in range(LAYERS):
        agg = _sc_msg(hw, src, dst, ew[l], zeros)
        if l + 1 < LAYERS:
            h, hw = _node_update(h, agg, Wu[l], bu[l], Wm[l + 1])
        else:
            out = _final_head(h, agg, Wu[l], bu[l], w_att, Wl1, bl1, Wl2, bl2)
    return out


# speedup vs baseline: 3.5850x; 3.5850x over previous
import functools

import jax
import jax.numpy as jnp
from jax import lax
from jax.experimental import pallas as pl
from jax.experimental.pallas import tpu as pltpu
from jax.experimental.pallas import tpu_sc as plsc

N = 10000
NP = 10240
E = 320000
NODE_DIM = 128
EDGE_DIM = 16
H = 64
LAYERS = 4

NC = 2
NS = 16
NW = NC * NS
C = 128
NCHUNK = E // C
CHUNKS_LO = NCHUNK // NW
CHUNKS_REM = NCHUNK - CHUNKS_LO * NW
ROWS_PER_TILE = NP // NS

_LRELU = 0.01


def _embed_body(nf, wn, bn, wm0, h_out, hw_out):
    h = jnp.dot(nf[...], wn[...], preferred_element_type=jnp.float32) + bn[...]
    h_out[...] = h
    hw_out[...] = jnp.dot(h, wm0[...], preferred_element_type=jnp.float32)


def _node_embed(node_feats, W_node, b_node, Wm0):
    blk = 2048
    grid = NP // blk
    return pl.pallas_call(
        _embed_body,
        grid=(grid,),
        in_specs=[
            pl.BlockSpec((blk, NODE_DIM), lambda i: (i, 0)),
            pl.BlockSpec((NODE_DIM, H), lambda i: (0, 0)),
            pl.BlockSpec((1, H), lambda i: (0, 0)),
            pl.BlockSpec((H, H), lambda i: (0, 0)),
        ],
        out_specs=[
            pl.BlockSpec((blk, H), lambda i: (i, 0)),
            pl.BlockSpec((blk, H), lambda i: (i, 0)),
        ],
        out_shape=[
            jax.ShapeDtypeStruct((NP, H), jnp.float32),
            jax.ShapeDtypeStruct((NP, H), jnp.float32),
        ],
    )(node_feats, W_node, b_node.reshape(1, H), Wm0)


def _edge_body(ef, weff, beff, o0, o1, o2, o3):
    y = jnp.dot(ef[...], weff[...], preferred_element_type=jnp.float32) + beff[...]
    o0[...] = y[:, 0 * H:1 * H]
    o1[...] = y[:, 1 * H:2 * H]
    o2[...] = y[:, 2 * H:3 * H]
    o3[...] = y[:, 3 * H:4 * H]


def _edge_terms(edge_feats, Weff, beff):
    blk = 8000
    grid = E // blk
    return pl.pallas_call(
        _edge_body,
        grid=(grid,),
        in_specs=[
            pl.BlockSpec((blk, EDGE_DIM), lambda i: (i, 0)),
            pl.BlockSpec((EDGE_DIM, LAYERS * H), lambda i: (0, 0)),
            pl.BlockSpec((1, LAYERS * H), lambda i: (0, 0)),
        ],
        out_specs=[pl.BlockSpec((blk, H), lambda i: (i, 0))] * LAYERS,
        out_shape=[jax.ShapeDtypeStruct((E, H), jnp.float32)] * LAYERS,
    )(edge_feats, Weff, beff.reshape(1, LAYERS * H))


def _update_body(h, agg, wu, bu, wm_next, h_out, hw_out):
    a = agg[0] + agg[1]
    x = jnp.dot(h[...] + a, wu[...], preferred_element_type=jnp.float32) + bu[...]
    hn = jnp.maximum(x, x * _LRELU)
    h_out[...] = hn
    hw_out[...] = jnp.dot(hn, wm_next[...], preferred_element_type=jnp.float32)


def _node_update(h, agg, Wu_l, bu_l, Wm_next):
    return pl.pallas_call(
        _update_body,
        out_shape=[
            jax.ShapeDtypeStruct((NP, H), jnp.float32),
            jax.ShapeDtypeStruct((NP, H), jnp.float32),
        ],
    )(h, agg, Wu_l, bu_l.reshape(1, H), Wm_next)


def _final_body(h, agg, wu, bu, watt, wl1, bl1, wl2, bl2, out):
    a = agg[0] + agg[1]
    x = jnp.dot(h[...] + a, wu[...], preferred_element_type=jnp.float32) + bu[...]
    hn = jnp.maximum(x, x * _LRELU)
    logits = jnp.dot(hn, watt[...], preferred_element_type=jnp.float32)
    rows = lax.broadcasted_iota(jnp.int32, (NP, 1), 0)
    logits = jnp.where(rows < N, logits, -jnp.inf)
    m = jnp.max(logits)
    p = jnp.exp(logits - m)
    attn = p / jnp.sum(p)
    sup = jnp.sum(attn * hn, axis=0, keepdims=True)
    z = jnp.dot(sup, wl1[...], preferred_element_type=jnp.float32) + bl1[...]
    z = jnp.maximum(z, 0.0)
    out[...] = jnp.dot(z, wl2[...], preferred_element_type=jnp.float32) + bl2[...]


def _final_head(h, agg, Wu_l, bu_l, w_att, Wl1, bl1, Wl2, bl2):
    return pl.pallas_call(
        _final_body,
        out_shape=jax.ShapeDtypeStruct((1, 1), jnp.float32),
    )(h, agg, Wu_l, bu_l.reshape(1, H), w_att.reshape(H, 1),
      Wl1, bl1.reshape(1, H), Wl2, bl2.reshape(1, 1))


def _sc_msg_body(hw_hbm, src_hbm, dst_hbm, ew_hbm, zeros_hbm, agg_hbm,
                 src_v, dst_v, rows_v, ew_v, agg_sh, sem_g, sem_e):
    cid = lax.axis_index("c")
    sid = lax.axis_index("s")
    wid = sid * NC + cid

    pltpu.sync_copy(zeros_hbm.at[pl.ds(sid * ROWS_PER_TILE, ROWS_PER_TILE)],
                    agg_sh.at[pl.ds(sid * ROWS_PER_TILE, ROWS_PER_TILE)])
    plsc.subcore_barrier()

    nloc = jnp.where(wid < CHUNKS_REM, CHUNKS_LO + 1, CHUNKS_LO)

    def chunk_body(i, carry):
        base = (wid + i * NW) * C
        pltpu.sync_copy(src_hbm.at[pl.ds(base, C)], src_v)
        pltpu.sync_copy(dst_hbm.at[pl.ds(base, C)], dst_v)
        ge = pltpu.async_copy(ew_hbm.at[pl.ds(base, C)], ew_v, sem_e)
        gg = pltpu.async_copy(hw_hbm.at[src_v], rows_v, sem_g)
        ge.wait()
        gg.wait()

        def row_body(r, rc):
            for k in range(H // 16):
                sl = pl.ds(k * 16, 16)
                x = rows_v[r, sl] + ew_v[r, sl]
                rows_v[r, sl] = jnp.maximum(x, x * _LRELU)
            return rc
        lax.fori_loop(0, C, row_body, 0)

        pltpu.sync_copy(rows_v, agg_sh.at[dst_v], add=True)
        return carry

    lax.fori_loop(0, nloc, chunk_body, 0)

    plsc.subcore_barrier()
    pltpu.sync_copy(agg_sh.at[pl.ds(sid * ROWS_PER_TILE, ROWS_PER_TILE)],
                    agg_hbm.at[cid, pl.ds(sid * ROWS_PER_TILE, ROWS_PER_TILE)])


@functools.cache
def _sc_msg_kernel():
    return pl.kernel(
        _sc_msg_body,
        out_type=jax.ShapeDtypeStruct((NC, NP, H), jnp.float32),
        mesh=plsc.VectorSubcoreMesh(core_axis_name="c", subcore_axis_name="s"),
        compiler_params=pltpu.CompilerParams(use_tc_tiling_on_sc=False),
        scratch_types=[
            pltpu.VMEM((C,), jnp.int32),
            pltpu.VMEM((C,), jnp.int32),
            pltpu.VMEM((C, H), jnp.float32),
            pltpu.VMEM((C, H), jnp.float32),
            pltpu.VMEM_SHARED((NP, H), jnp.float32),
            pltpu.SemaphoreType.DMA,
            pltpu.SemaphoreType.DMA,
        ],
    )


def _sc_msg(*args):
    return _sc_msg_kernel()(*args)


def kernel(graph, node_feats, edge_feats, W_node, b_node, W_edge, b_edge,
           Wm, bm, Wu, bu, w_att, Wl1, bl1, Wl2, bl2):
    src = graph[0]
    dst = graph[1]

    Weff = jnp.concatenate([W_edge @ Wm[l] for l in range(LAYERS)], axis=1)
    beff = jnp.concatenate([b_edge @ Wm[l] + bm[l] for l in range(LAYERS)])

    node_feats_p = jnp.pad(node_feats, ((0, NP - N), (0, 0)))
    h, hw = _node_embed(node_feats_p, W_node, b_node, Wm[0])
    ew = _edge_terms(edge_feats, Weff, beff)
    zeros = jnp.zeros((NP, H), jnp.float32)

    out = None
    for l in range(LAYERS):
        agg = _sc_msg(hw, src, dst, ew[l], zeros)
        if l + 1 < LAYERS:
            h, hw = _node_update(h, agg, Wu[l], bu[l], Wm[l + 1])
        else:
            out = _final_head(h, agg, Wu[l], bu[l], w_att, Wl1, bl1, Wl2, bl2)
    return out
